# baseline (speedup 1.0000x reference)
import functools

import jax
import jax.numpy as jnp
from jax import lax
from jax.experimental import pallas as pl
from jax.experimental.pallas import tpu as pltpu
from jax.experimental.pallas import tpu_sc as plsc

N = 2048
L = 16
NC, NS = 2, 16
NW = NC * NS
ROWS_PER_W = N // NW
NCHUNK = N // L
NFINE = 64
NCOARSE = 16

_MESH = plsc.VectorSubcoreMesh(
    core_axis_name="c", subcore_axis_name="s", num_cores=NC, num_subcores=NS
)


@functools.partial(
    pl.kernel,
    out_type=jax.ShapeDtypeStruct((NW, ROWS_PER_W * NCOARSE), jnp.float32),
    mesh=_MESH,
    compiler_params=pltpu.CompilerParams(needs_layout_passes=False),
    scratch_types=[
        pltpu.VMEM((2 * N,), jnp.float32),
        pltpu.VMEM((N,), jnp.float32),
        pltpu.VMEM((N,), jnp.float32),
        pltpu.VMEM((NFINE * NCOARSE,), jnp.float32),
        pltpu.VMEM((ROWS_PER_W * NCOARSE,), jnp.float32),
        pltpu.VMEM((32,), jnp.int32),
        pltpu.VMEM((32,), jnp.int32),
    ],
)
def _sc_grid_kernel(pos_hbm, out_hbm, pos_v, x_v, y_v, occ_v, grid_v, lutx_v, luty_v):
    wid = lax.axis_index("s") * NC + lax.axis_index("c")
    base = wid * ROWS_PER_W
    pltpu.sync_copy(pos_hbm, pos_v)

    jbase = lax.iota(jnp.int32, L)
    ones = jnp.ones((L,), jnp.float32)
    zeros = jnp.zeros((L,), jnp.float32)

    @plsc.parallel_loop(0, N, step=L, unroll=8)
    def deint(j0):
        idx = (jbase + j0) << 1
        x_v[pl.ds(j0, L)] = plsc.load_gather(pos_v, [idx]) * 4.0
        y_v[pl.ds(j0, L)] = plsc.load_gather(pos_v, [idx + 1]) * 4.0

    for k in range(2):
        f = jbase + k * L
        lutx_v[pl.ds(k * L, L)] = ((f & 7) << 7) | ((f >> 3) << 2)
        luty_v[pl.ds(k * L, L)] = ((f & 7) << 4) | (f >> 3)

    for k in range(NFINE):
        occ_v[pl.ds(k * L, L)] = zeros

    def row_body(r, row_carry):
        i = base + r
        ibc = jnp.full((L,), i, jnp.int32)
        xi = plsc.load_gather(x_v, [ibc])
        yi = plsc.load_gather(y_v, [ibc])
        plsc.store_scatter(x_v, [ibc], jnp.full((L,), 1e9, jnp.float32))

        @plsc.parallel_loop(0, N, step=L, unroll=8)
        def j_body(j0):
            dx = x_v[pl.ds(j0, L)] - xi
            dy = y_v[pl.ds(j0, L)] - yi
            tx = dx + 16.0
            ty = dy + 16.0
            fx = tx.astype(jnp.int32)
            fy = ty.astype(jnp.int32)
            ub = (
                lax.bitcast_convert_type(fx, jnp.uint32)
                | lax.bitcast_convert_type(fy, jnp.uint32)
            ) < jnp.uint32(32)
            m = ((dx >= -16.0) & (dy >= -16.0)) & ub
            cell = plsc.load_gather(lutx_v, [fx], mask=m) + plsc.load_gather(
                luty_v, [fy], mask=m
            )
            plsc.store_scatter(occ_v, [cell], ones, mask=m)

        plsc.store_scatter(x_v, [ibc], xi)

        accs = []
        for k in range(NFINE):
            v = occ_v[pl.ds(k * L, L)]
            occ_v[pl.ds(k * L, L)] = zeros
            if k < 8:
                accs.append(v)
            else:
                accs[k % 8] = accs[k % 8] + v
        while len(accs) > 1:
            accs = [accs[a] + accs[a + 1] for a in range(0, len(accs), 2)]
        grid_v[pl.ds(r * NCOARSE, NCOARSE)] = accs[0]
        return row_carry

    lax.fori_loop(0, ROWS_PER_W, row_body, None)
    pltpu.sync_copy(grid_v, out_hbm.at[wid])


def _tc_linear_body(g_ref, w_ref, b_ref, o_ref):
    acc = lax.dot_general(
        g_ref[...], w_ref[...], (((1,), (1,)), ((), ())),
        preferred_element_type=jnp.float32,
    )
    o_ref[...] = jnp.maximum(acc + b_ref[...], 0.0)


def kernel(hidden_state, obs1, obs2, W, b):
    grid = _sc_grid_kernel(obs2.reshape(-1)).reshape(N, NCOARSE)
    out = pl.pallas_call(
        _tc_linear_body,
        out_shape=jax.ShapeDtypeStruct((N, W.shape[0]), jnp.float32),
    )(grid, W, b.reshape(1, -1))
    return out


# speedup vs baseline: 1.0766x; 1.0766x over previous
import functools

import jax
import jax.numpy as jnp
from jax import lax
from jax.experimental import pallas as pl
from jax.experimental.pallas import tpu as pltpu
from jax.experimental.pallas import tpu_sc as plsc

N = 2048
L = 16
NC, NS = 2, 16
NW = NC * NS
ROWS_PER_W = N // NW
NCHUNK = N // L
NFINE = 64
NCOARSE = 16

_MESH = plsc.VectorSubcoreMesh(
    core_axis_name="c", subcore_axis_name="s", num_cores=NC, num_subcores=NS
)


@functools.partial(
    pl.kernel,
    out_type=jax.ShapeDtypeStruct((NW, ROWS_PER_W * NCOARSE), jnp.float32),
    mesh=_MESH,
    compiler_params=pltpu.CompilerParams(needs_layout_passes=False),
    scratch_types=[
        pltpu.VMEM((2 * N,), jnp.float32),
        pltpu.VMEM((N,), jnp.float32),
        pltpu.VMEM((N,), jnp.float32),
        pltpu.VMEM((NFINE * NCOARSE,), jnp.float32),
        pltpu.VMEM((ROWS_PER_W * NCOARSE,), jnp.float32),
        pltpu.VMEM((32,), jnp.int32),
        pltpu.VMEM((32,), jnp.int32),
    ],
)
def _sc_grid_kernel(pos_hbm, out_hbm, pos_v, x_v, y_v, occ_v, grid_v, lutx_v, luty_v):
    wid = lax.axis_index("s") * NC + lax.axis_index("c")
    base = wid * ROWS_PER_W
    pltpu.sync_copy(pos_hbm, pos_v)

    jbase = lax.iota(jnp.int32, L)
    ones = jnp.ones((L,), jnp.float32)
    zeros = jnp.zeros((L,), jnp.float32)

    @plsc.parallel_loop(0, N, step=L, unroll=8)
    def deint(j0):
        idx = (jbase + j0) << 1
        x_v[pl.ds(j0, L)] = plsc.load_gather(pos_v, [idx]) * 4.0
        y_v[pl.ds(j0, L)] = plsc.load_gather(pos_v, [idx + 1]) * 4.0

    for k in range(2):
        f = jbase + k * L
        lutx_v[pl.ds(k * L, L)] = ((f & 7) << 7) | ((f >> 3) << 2)
        luty_v[pl.ds(k * L, L)] = ((f & 7) << 4) | (f >> 3)

    for k in range(NFINE):
        occ_v[pl.ds(k * L, L)] = zeros

    def row_body(r, row_carry):
        i = base + r
        ibc = jnp.full((L,), i, jnp.int32)
        xi = plsc.load_gather(x_v, [ibc])
        yi = plsc.load_gather(y_v, [ibc])
        plsc.store_scatter(x_v, [ibc], jnp.full((L,), 1e9, jnp.float32))

        @plsc.parallel_loop(0, N, step=L, unroll=8)
        def j_body(j0):
            dx = x_v[pl.ds(j0, L)] - xi
            dy = y_v[pl.ds(j0, L)] - yi
            tx = dx + 16.0
            ty = dy + 16.0
            fx = tx.astype(jnp.int32)
            fy = ty.astype(jnp.int32)
            ub = (
                lax.bitcast_convert_type(fx, jnp.uint32)
                | lax.bitcast_convert_type(fy, jnp.uint32)
            ) < jnp.uint32(32)
            m = (jnp.minimum(dx, dy) >= -16.0) & ub
            cell = plsc.load_gather(lutx_v, [fx], mask=m) + plsc.load_gather(
                luty_v, [fy], mask=m
            )
            plsc.store_scatter(occ_v, [cell], ones, mask=m)

        plsc.store_scatter(x_v, [ibc], xi)

        accs = []
        for k in range(NFINE):
            v = occ_v[pl.ds(k * L, L)]
            occ_v[pl.ds(k * L, L)] = zeros
            if k < 8:
                accs.append(v)
            else:
                accs[k % 8] = accs[k % 8] + v
        while len(accs) > 1:
            accs = [accs[a] + accs[a + 1] for a in range(0, len(accs), 2)]
        grid_v[pl.ds(r * NCOARSE, NCOARSE)] = accs[0]
        return row_carry

    lax.fori_loop(0, ROWS_PER_W, row_body, None)
    pltpu.sync_copy(grid_v, out_hbm.at[wid])


def _tc_linear_body(g_ref, w_ref, b_ref, o_ref):
    acc = lax.dot_general(
        g_ref[...], w_ref[...], (((1,), (1,)), ((), ())),
        preferred_element_type=jnp.float32,
    )
    o_ref[...] = jnp.maximum(acc + b_ref[...], 0.0)


def kernel(hidden_state, obs1, obs2, W, b):
    grid = _sc_grid_kernel(obs2.reshape(-1)).reshape(N, NCOARSE)
    out = pl.pallas_call(
        _tc_linear_body,
        out_shape=jax.ShapeDtypeStruct((N, W.shape[0]), jnp.float32),
    )(grid, W, b.reshape(1, -1))
    return out
